# baseline (speedup 1.0000x reference)
import functools

import jax
import jax.numpy as jnp
from jax.experimental import pallas as pl
from jax.experimental.pallas import tpu as pltpu


def _build_patches(x_ref, p_ref, *, cin, kh, kw, w_img, n_lanes, hw):
    for ikh in range(kh):
        for ikw in range(kw):
            t = ikh * kw + ikw
            off = ikh * w_img + ikw
            n = min(n_lanes, hw - off)
            p_ref[t * cin:(t + 1) * cin, :n] = (
                x_ref[0, :, off:off + n].astype(jnp.bfloat16))


def _conv_stats_kernel(w_ref, x_ref, stats_ref, p_ref, *,
                       cin, kh, kw, w_img, wo, n_lanes, hw):
    _build_patches(x_ref, p_ref, cin=cin, kh=kh, kw=kw, w_img=w_img,
                   n_lanes=n_lanes, hw=hw)
    y = jnp.dot(w_ref[...], p_ref[...], preferred_element_type=jnp.float32)
    lane = jax.lax.broadcasted_iota(jnp.int32, (1, n_lanes), 1)
    ym = jnp.where(lane % w_img < wo, y, 0.0)
    stats_ref[0, :, 0:1] = jnp.sum(ym, axis=1, keepdims=True)
    stats_ref[0, :, 1:2] = jnp.sum(ym * ym, axis=1, keepdims=True)


def _conv_norm_kernel(w_ref, stats_ref, x_ref, o_ref, p_ref, *,
                      cin, kh, kw, w_img, wo, ho, n_lanes, hw, count, eps):
    _build_patches(x_ref, p_ref, cin=cin, kh=kh, kw=kw, w_img=w_img,
                   n_lanes=n_lanes, hw=hw)
    y = jnp.dot(w_ref[...], p_ref[...], preferred_element_type=jnp.float32)
    st = jnp.sum(stats_ref[...], axis=0)
    s = st[:, 0:1]
    ss = st[:, 1:2]
    mean = s * (1.0 / count)
    var = (ss - s * mean) * (1.0 / (count - 1.0))
    inv = 1.0 / (jnp.sqrt(var) + eps)
    o_ref[0] = (y - mean) * inv


def _compact_kernel(xp_ref, o_ref, *, wo):
    o_ref[0] = xp_ref[0, :, :, :wo]


def kernel(X, conv_weight):
    n, cin, h, w_img = X.shape
    cout, _, kh, kw = conv_weight.shape
    ho = h - kh + 1
    wo = w_img - kw + 1
    hw = h * w_img
    n_lanes = ho * w_img
    k_dim = cin * kh * kw
    count = float(n * ho * wo)
    eps = 1.0

    x3 = X.reshape(n, cin, hw)
    w_mat = (conv_weight.transpose(0, 2, 3, 1)
             .reshape(cout, k_dim).astype(jnp.bfloat16))

    vmem_limit = 48 * 1024 * 1024

    stats = pl.pallas_call(
        functools.partial(_conv_stats_kernel, cin=cin, kh=kh, kw=kw,
                          w_img=w_img, wo=wo, n_lanes=n_lanes, hw=hw),
        out_shape=jax.ShapeDtypeStruct((n, cout, 2), jnp.float32),
        grid=(n,),
        in_specs=[pl.BlockSpec((cout, k_dim), lambda i: (0, 0)),
                  pl.BlockSpec((1, cin, hw), lambda i: (i, 0, 0))],
        out_specs=pl.BlockSpec((1, cout, 2), lambda i: (i, 0, 0)),
        scratch_shapes=[pltpu.VMEM((k_dim, n_lanes), jnp.bfloat16)],
        compiler_params=pltpu.CompilerParams(
            dimension_semantics=("parallel",),
            vmem_limit_bytes=vmem_limit),
    )(w_mat, x3)

    out_pad = pl.pallas_call(
        functools.partial(_conv_norm_kernel, cin=cin, kh=kh, kw=kw,
                          w_img=w_img, wo=wo, ho=ho, n_lanes=n_lanes, hw=hw,
                          count=count, eps=eps),
        out_shape=jax.ShapeDtypeStruct((n, cout, n_lanes), jnp.float32),
        grid=(n,),
        in_specs=[pl.BlockSpec((cout, k_dim), lambda i: (0, 0)),
                  pl.BlockSpec((n, cout, 2), lambda i: (0, 0, 0)),
                  pl.BlockSpec((1, cin, hw), lambda i: (i, 0, 0))],
        out_specs=pl.BlockSpec((1, cout, n_lanes), lambda i: (i, 0, 0)),
        scratch_shapes=[pltpu.VMEM((k_dim, n_lanes), jnp.bfloat16)],
        compiler_params=pltpu.CompilerParams(
            dimension_semantics=("parallel",),
            vmem_limit_bytes=vmem_limit),
    )(w_mat, stats, x3)
    out = pl.pallas_call(
        functools.partial(_compact_kernel, wo=wo),
        out_shape=jax.ShapeDtypeStruct((n, cout, ho, wo), jnp.float32),
        grid=(n,),
        in_specs=[pl.BlockSpec((1, cout, ho, w_img), lambda i: (i, 0, 0, 0))],
        out_specs=pl.BlockSpec((1, cout, ho, wo), lambda i: (i, 0, 0, 0)),
        compiler_params=pltpu.CompilerParams(
            dimension_semantics=("parallel",),
            vmem_limit_bytes=vmem_limit),
    )(out_pad.reshape(n, cout, ho, w_img))
    return out


# speedup vs baseline: 1.6012x; 1.6012x over previous
import functools

import jax
import jax.numpy as jnp
from jax.experimental import pallas as pl
from jax.experimental.pallas import tpu as pltpu


def _build_patches(x_ref, p_ref, *, cin, kh, kw, w_img, n_lanes, hw):
    for ikh in range(kh):
        for ikw in range(kw):
            t = ikh * kw + ikw
            off = ikh * w_img + ikw
            n = min(n_lanes, hw - off)
            p_ref[t * cin:(t + 1) * cin, :n] = (
                x_ref[0, :, off:off + n].astype(jnp.bfloat16))


def _fused_kernel(w_ref, x_ref, o_ref, p_ref, acc_ref, *,
                  cin, kh, kw, w_img, wo, n_lanes, hw, count, eps):
    ph = pl.program_id(0)
    i = pl.program_id(1)
    _build_patches(x_ref, p_ref, cin=cin, kh=kh, kw=kw, w_img=w_img,
                   n_lanes=n_lanes, hw=hw)
    y = jnp.dot(w_ref[...], p_ref[...], preferred_element_type=jnp.float32)

    @pl.when(jnp.logical_and(ph == 0, i == 0))
    def _():
        acc_ref[...] = jnp.zeros_like(acc_ref)

    @pl.when(ph == 0)
    def _():
        lane = jax.lax.broadcasted_iota(jnp.int32, (1, n_lanes), 1)
        ym = jnp.where(lane % w_img < wo, y, 0.0)
        acc_ref[:, 0:1] += jnp.sum(ym, axis=1, keepdims=True)
        acc_ref[:, 1:2] += jnp.sum(ym * ym, axis=1, keepdims=True)

    @pl.when(ph == 1)
    def _():
        s = acc_ref[:, 0:1]
        ss = acc_ref[:, 1:2]
        mean = s * (1.0 / count)
        var = (ss - s * mean) * (1.0 / (count - 1.0))
        inv = 1.0 / (jnp.sqrt(var) + eps)
        o_ref[0] = (y - mean) * inv


def kernel(X, conv_weight):
    n, cin, h, w_img = X.shape
    cout, _, kh, kw = conv_weight.shape
    ho = h - kh + 1
    wo = w_img - kw + 1
    hw = h * w_img
    n_lanes = ho * w_img
    k_dim = cin * kh * kw
    count = float(n * ho * wo)
    eps = 1.0

    x3 = X.reshape(n, cin, hw)
    w_mat = (conv_weight.transpose(0, 2, 3, 1)
             .reshape(cout, k_dim).astype(jnp.bfloat16))

    out_pad = pl.pallas_call(
        functools.partial(_fused_kernel, cin=cin, kh=kh, kw=kw,
                          w_img=w_img, wo=wo, n_lanes=n_lanes, hw=hw,
                          count=count, eps=eps),
        out_shape=jax.ShapeDtypeStruct((n, cout, n_lanes), jnp.float32),
        grid=(2, n),
        in_specs=[pl.BlockSpec((cout, k_dim), lambda p, i: (0, 0)),
                  pl.BlockSpec((1, cin, hw), lambda p, i: (i, 0, 0))],
        out_specs=pl.BlockSpec((1, cout, n_lanes), lambda p, i: (p * i, 0, 0)),
        scratch_shapes=[pltpu.VMEM((k_dim, n_lanes), jnp.bfloat16),
                        pltpu.VMEM((cout, 2), jnp.float32)],
        compiler_params=pltpu.CompilerParams(
            dimension_semantics=("arbitrary", "arbitrary"),
            vmem_limit_bytes=48 * 1024 * 1024),
    )(w_mat, x3)
    return out_pad.reshape(n, cout, ho, w_img)[:, :, :, :wo]


# speedup vs baseline: 1.8459x; 1.1528x over previous
import functools

import jax
import jax.numpy as jnp
from jax.experimental import pallas as pl
from jax.experimental.pallas import tpu as pltpu


def _build_patches(x_ref, p_ref, *, cin, kh, kw, w_img, n_lanes, hw):
    for ikh in range(kh):
        for ikw in range(kw):
            t = ikh * kw + ikw
            off = ikh * w_img + ikw
            n = min(n_lanes, hw - off)
            p_ref[t * cin:(t + 1) * cin, :n] = (
                x_ref[0, :, off:off + n].astype(jnp.bfloat16))


def _conv_stats_kernel(w_ref, x_ref, stats_ref, p_ref, *,
                       cin, kh, kw, w_img, wo, n_lanes, hw):
    _build_patches(x_ref, p_ref, cin=cin, kh=kh, kw=kw, w_img=w_img,
                   n_lanes=n_lanes, hw=hw)
    y = jnp.dot(w_ref[...], p_ref[...], preferred_element_type=jnp.float32)
    lane = jax.lax.broadcasted_iota(jnp.int32, (1, n_lanes), 1)
    ym = jnp.where(lane % w_img < wo, y, 0.0)
    stats_ref[0, :, 0:1] = jnp.sum(ym, axis=1, keepdims=True)
    stats_ref[0, :, 1:2] = jnp.sum(ym * ym, axis=1, keepdims=True)


def _conv_norm_kernel(w_ref, stats_ref, x_ref, o_ref, p_ref, *,
                      cin, kh, kw, w_img, wo, ho, n_lanes, hw, count, eps):
    _build_patches(x_ref, p_ref, cin=cin, kh=kh, kw=kw, w_img=w_img,
                   n_lanes=n_lanes, hw=hw)
    y = jnp.dot(w_ref[...], p_ref[...], preferred_element_type=jnp.float32)
    st = jnp.sum(stats_ref[...], axis=0)
    s = st[:, 0:1]
    ss = st[:, 1:2]
    mean = s * (1.0 / count)
    var = (ss - s * mean) * (1.0 / (count - 1.0))
    inv = 1.0 / (jnp.sqrt(var) + eps)
    o = (y - mean) * inv
    o_ref[0] = jnp.concatenate(
        [o[:, r * w_img:r * w_img + wo] for r in range(ho)], axis=1)


def kernel(X, conv_weight):
    n, cin, h, w_img = X.shape
    cout, _, kh, kw = conv_weight.shape
    ho = h - kh + 1
    wo = w_img - kw + 1
    hw = h * w_img
    n_lanes = ho * w_img
    k_dim = cin * kh * kw
    count = float(n * ho * wo)
    eps = 1.0

    x3 = X.reshape(n, cin, hw)
    w_mat = (conv_weight.transpose(0, 2, 3, 1)
             .reshape(cout, k_dim).astype(jnp.bfloat16))

    vmem_limit = 48 * 1024 * 1024

    stats = pl.pallas_call(
        functools.partial(_conv_stats_kernel, cin=cin, kh=kh, kw=kw,
                          w_img=w_img, wo=wo, n_lanes=n_lanes, hw=hw),
        out_shape=jax.ShapeDtypeStruct((n, cout, 2), jnp.float32),
        grid=(n,),
        in_specs=[pl.BlockSpec((cout, k_dim), lambda i: (0, 0)),
                  pl.BlockSpec((1, cin, hw), lambda i: (i, 0, 0))],
        out_specs=pl.BlockSpec((1, cout, 2), lambda i: (i, 0, 0)),
        scratch_shapes=[pltpu.VMEM((k_dim, n_lanes), jnp.bfloat16)],
        compiler_params=pltpu.CompilerParams(
            dimension_semantics=("parallel",),
            vmem_limit_bytes=vmem_limit),
    )(w_mat, x3)

    out_flat = pl.pallas_call(
        functools.partial(_conv_norm_kernel, cin=cin, kh=kh, kw=kw,
                          w_img=w_img, wo=wo, ho=ho, n_lanes=n_lanes, hw=hw,
                          count=count, eps=eps),
        out_shape=jax.ShapeDtypeStruct((n, cout, ho * wo), jnp.float32),
        grid=(n,),
        in_specs=[pl.BlockSpec((cout, k_dim), lambda i: (0, 0)),
                  pl.BlockSpec((n, cout, 2), lambda i: (0, 0, 0)),
                  pl.BlockSpec((1, cin, hw), lambda i: (i, 0, 0))],
        out_specs=pl.BlockSpec((1, cout, ho * wo), lambda i: (i, 0, 0)),
        scratch_shapes=[pltpu.VMEM((k_dim, n_lanes), jnp.bfloat16)],
        compiler_params=pltpu.CompilerParams(
            dimension_semantics=("parallel",),
            vmem_limit_bytes=vmem_limit),
    )(w_mat, stats, x3)
    return out_flat.reshape(n, cout, ho, wo)


# speedup vs baseline: 2.0674x; 1.1200x over previous
import functools

import jax
import jax.numpy as jnp
from jax.experimental import pallas as pl
from jax.experimental.pallas import tpu as pltpu


def _build_patches_t(x_ref, p_ref, *, cin, kh, kw, w_img, n_rows, hw):
    for ikh in range(kh):
        for ikw in range(kw):
            t = ikh * kw + ikw
            off = ikh * w_img + ikw
            m = min(n_rows, hw - off)
            p_ref[:m, t * cin:(t + 1) * cin] = (
                x_ref[0, off:off + m, :].astype(jnp.bfloat16))


def _conv_stats_kernel(w_ref, x_ref, stats_ref, p_ref, *,
                       cin, kh, kw, w_img, wo, n_rows, hw):
    _build_patches_t(x_ref, p_ref, cin=cin, kh=kh, kw=kw, w_img=w_img,
                     n_rows=n_rows, hw=hw)
    y = jnp.dot(p_ref[...], w_ref[...], preferred_element_type=jnp.float32)
    row = jax.lax.broadcasted_iota(jnp.int32, (n_rows, 1), 0)
    ym = jnp.where(row % w_img < wo, y, 0.0)
    stats_ref[0, 0:1, :] = jnp.sum(ym, axis=0, keepdims=True)
    stats_ref[0, 1:2, :] = jnp.sum(ym * ym, axis=0, keepdims=True)


def _conv_norm_kernel(w_ref, stats_ref, x_ref, o_hbm, p_ref, yc_ref, sem, *,
                      cin, kh, kw, w_img, wo, ho, n_rows, hw, count, eps):
    i = pl.program_id(0)
    _build_patches_t(x_ref, p_ref, cin=cin, kh=kh, kw=kw, w_img=w_img,
                     n_rows=n_rows, hw=hw)
    y = jnp.dot(p_ref[...], w_ref[...], preferred_element_type=jnp.float32)
    st = jnp.sum(stats_ref[...], axis=0)
    s = st[0:1, :]
    ss = st[1:2, :]
    mean = s * (1.0 / count)
    var = (ss - s * mean) * (1.0 / (count - 1.0))
    inv = 1.0 / (jnp.sqrt(var) + eps)
    o = (y - mean) * inv
    yc_ref[...] = jnp.concatenate(
        [o[r * w_img:r * w_img + wo, :] for r in range(ho)], axis=0)
    cp = pltpu.make_async_copy(yc_ref, o_hbm.at[:, i, :], sem)
    cp.start()
    cp.wait()


def kernel(X, conv_weight):
    n, cin, h, w_img = X.shape
    cout, _, kh, kw = conv_weight.shape
    ho = h - kh + 1
    wo = w_img - kw + 1
    hw = h * w_img
    n_rows = ho * w_img
    k_dim = cin * kh * kw
    count = float(n * ho * wo)
    eps = 1.0

    xt = X.transpose(0, 2, 3, 1).reshape(n, hw, cin)
    w_t = (conv_weight.transpose(2, 3, 1, 0)
           .reshape(k_dim, cout).astype(jnp.bfloat16))

    vmem_limit = 48 * 1024 * 1024

    stats = pl.pallas_call(
        functools.partial(_conv_stats_kernel, cin=cin, kh=kh, kw=kw,
                          w_img=w_img, wo=wo, n_rows=n_rows, hw=hw),
        out_shape=jax.ShapeDtypeStruct((n, 2, cout), jnp.float32),
        grid=(n,),
        in_specs=[pl.BlockSpec((k_dim, cout), lambda i: (0, 0)),
                  pl.BlockSpec((1, hw, cin), lambda i: (i, 0, 0))],
        out_specs=pl.BlockSpec((1, 2, cout), lambda i: (i, 0, 0)),
        scratch_shapes=[pltpu.VMEM((n_rows, k_dim), jnp.bfloat16)],
        compiler_params=pltpu.CompilerParams(
            dimension_semantics=("parallel",),
            vmem_limit_bytes=vmem_limit),
    )(w_t, xt)

    out3 = pl.pallas_call(
        functools.partial(_conv_norm_kernel, cin=cin, kh=kh, kw=kw,
                          w_img=w_img, wo=wo, ho=ho, n_rows=n_rows, hw=hw,
                          count=count, eps=eps),
        out_shape=jax.ShapeDtypeStruct((ho * wo, n, cout), jnp.float32),
        grid=(n,),
        in_specs=[pl.BlockSpec((k_dim, cout), lambda i: (0, 0)),
                  pl.BlockSpec((n, 2, cout), lambda i: (0, 0, 0)),
                  pl.BlockSpec((1, hw, cin), lambda i: (i, 0, 0))],
        out_specs=pl.BlockSpec(memory_space=pl.ANY),
        scratch_shapes=[pltpu.VMEM((n_rows, k_dim), jnp.bfloat16),
                        pltpu.VMEM((ho * wo, cout), jnp.float32),
                        pltpu.SemaphoreType.DMA],
        compiler_params=pltpu.CompilerParams(
            dimension_semantics=("parallel",),
            vmem_limit_bytes=vmem_limit),
    )(w_t, stats, xt)
    return out3.reshape(ho, wo, n, cout).transpose(2, 3, 0, 1)
